# baseline (speedup 1.0000x reference)
import functools

import jax
import jax.numpy as jnp
from jax import lax
from jax.experimental import pallas as pl
from jax.experimental.pallas import tpu as pltpu
from jax.experimental.pallas import tpu_sc as plsc

f32 = jnp.float32

N = 50000
M = 12500
K = 16
NP = 50176
E = K * NP
NW = 32
CHUNK = 128
CH = E // (NW * CHUNK)
T = 512
GRID = NP // T
CNT_E = float(N * K)
CNT_N = float(N)


def _sc_gather(table, idx3, D, dtype=f32):
  mesh = plsc.VectorSubcoreMesh(core_axis_name="c", subcore_axis_name="s")

  @functools.partial(
      pl.kernel,
      out_type=jax.ShapeDtypeStruct((E, D), dtype),
      mesh=mesh,
      compiler_params=pltpu.CompilerParams(use_tc_tiling_on_sc=False),
      scratch_types=[
          pltpu.VMEM((CH, CHUNK), jnp.int32),
          pltpu.VMEM((CHUNK, D), dtype),
          pltpu.VMEM((CHUNK, D), dtype),
          pltpu.SemaphoreType.DMA,
          pltpu.SemaphoreType.DMA,
          pltpu.SemaphoreType.DMA,
          pltpu.SemaphoreType.DMA,
      ],
  )
  def gk(table_hbm, idx_hbm, out_hbm, idx_v, rows0, rows1, g0, g1, w0, w1):
    wid = lax.axis_index("s") * 2 + lax.axis_index("c")
    pltpu.sync_copy(idx_hbm.at[wid], idx_v)

    def body(i, carry):
      j0 = i * 2
      j1 = j0 + 1
      c0 = pltpu.async_copy(table_hbm.at[idx_v.at[j0]], rows0, g0)
      c1 = pltpu.async_copy(table_hbm.at[idx_v.at[j1]], rows1, g1)
      c0.wait()
      wc0 = pltpu.async_copy(
          rows0, out_hbm.at[pl.ds((wid * CH + j0) * CHUNK, CHUNK)], w0)
      c1.wait()
      wc1 = pltpu.async_copy(
          rows1, out_hbm.at[pl.ds((wid * CH + j1) * CHUNK, CHUNK)], w1)
      wc0.wait()
      wc1.wait()
      return carry

    lax.fori_loop(0, CH // 2, body, 0)

  return gk(table, idx3)


def _dot3(w_ref, x):
  return lax.dot_general(w_ref[...], x, (((1,), (0,)), ((), ())),
                         preferred_element_type=f32)


def _bnrelu3(x, bn_ref):
  c = x.shape[0]
  m = bn_ref[0].reshape(c, 1, 1)
  s = bn_ref[1].reshape(c, 1, 1)
  return jnp.maximum((x - m) * s, 0.0)


def _lanefold(x):
  r = x[..., 0:128]
  for o in range(128, x.shape[-1], 128):
    r = r + x[..., o:o + 128]
  return r


def _stat3(x, mask):
  xm = jnp.where(mask[None, :, :], x, 0.0)
  s = jnp.sum(_lanefold(xm), axis=1)
  xq = jnp.where(mask[None, :, :], x * x, 0.0)
  q = jnp.sum(_lanefold(xq), axis=1)
  return s, q


def _acc(acc_ref, part, i):
  @pl.when(i == 0)
  def _():
    acc_ref[...] = jnp.zeros_like(acc_ref[...])
  acc_ref[...] = acc_ref[...] + part


def _edge_mask(i):
  col = i * T + lax.broadcasted_iota(jnp.int32, (K, T), 1)
  return col < N


def _vi(loc, gn, q):
  rn = jnp.sqrt(loc[0] * loc[0] + loc[1] * loc[1] + loc[2] * loc[2])
  rinv = 1.0 / jnp.maximum(rn, 1e-12)
  rh = [l * rinv for l in loc]
  th2 = q[0] * rh[0] + q[1] * rh[1] + q[2] * rh[2]
  v = [q[c] - th2 * rh[c] for c in range(3)]
  vn = jnp.sqrt(v[0] * v[0] + v[1] * v[1] + v[2] * v[2])
  vinv = 1.0 / jnp.maximum(vn, 1e-12)
  vm = [x * vinv for x in v]
  w = [rh[1] * vm[2] - rh[2] * vm[1],
       rh[2] * vm[0] - rh[0] * vm[2],
       rh[0] * vm[1] - rh[1] * vm[0]]
  wnn = jnp.sqrt(w[0] * w[0] + w[1] * w[1] + w[2] * w[2])
  winv = 1.0 / jnp.maximum(wnn, 1e-12)
  wm = [x * winv for x in w]
  th1 = gn[0] * q[0] + gn[1] * q[1] + gn[2] * q[2]
  th3 = rh[0] * gn[0] + rh[1] * gn[1] + rh[2] * gn[2]
  th4 = loc[0] * q[0] + loc[1] * q[1] + loc[2] * q[2]
  th6 = gn[0] * vm[0] + gn[1] * vm[1] + gn[2] * vm[2]
  th7 = gn[0] * wm[0] + gn[1] * wm[1] + gn[2] * wm[2]
  cgq = [gn[1] * q[2] - gn[2] * q[1],
         gn[2] * q[0] - gn[0] * q[2],
         gn[0] * q[1] - gn[1] * q[0]]
  th8 = loc[0] * cgq[0] + loc[1] * cgq[1] + loc[2] * cgq[2]
  return [th1, th2, th3, th4, th3, th6, th7, th8, rn, loc[0], loc[1], loc[2]]


def _p1_body(g_ref, dx_ref, dn_ref, pe0_ref, wn0_ref, wni_ref, acc_ref):
  i = pl.program_id(0)
  dx = [jnp.broadcast_to(dx_ref[c][None, :], (K, T)) for c in range(3)]
  dn = [jnp.broadcast_to(dn_ref[c][None, :], (K, T)) for c in range(3)]
  loc = [g_ref[c] - dx[c] for c in range(3)]
  gn = [g_ref[3 + c] for c in range(3)]
  wni = _vi(loc, gn, dn)
  for c in range(12):
    wni_ref[c] = wni[c]
  x = wni_ref[...]
  pe0 = _dot3(pe0_ref, x[9:12])
  wn0 = _dot3(wn0_ref, x)
  mask = _edge_mask(i)
  s_pe, q_pe = _stat3(pe0, mask)
  s_wn, q_wn = _stat3(wn0, mask)
  _acc(acc_ref, jnp.concatenate([s_pe, s_wn, q_pe, q_wn], axis=0), i)


def _p2_body(wni_ref, pe0_ref, wn0_ref, pe1_ref, wn1_ref, bnpe0_ref,
             bnwn0_ref, acc_ref):
  i = pl.program_id(0)
  x = wni_ref[...]
  hpe = _bnrelu3(_dot3(pe0_ref, x[9:12]), bnpe0_ref)
  hwn = _bnrelu3(_dot3(wn0_ref, x), bnwn0_ref)
  pe1 = _dot3(pe1_ref, hpe)
  wn1 = _dot3(wn1_ref, hwn)
  mask = _edge_mask(i)
  s_pe, q_pe = _stat3(pe1, mask)
  s_wn, q_wn = _stat3(wn1, mask)
  _acc(acc_ref, jnp.concatenate([s_pe, s_wn, q_pe, q_wn], axis=0), i)


def _p3_body(wni_ref, pe0_ref, wn0_ref, pe1_ref, wn1_ref, wn2_ref, bnpe0_ref,
             bnwn0_ref, bnpe1_ref, bnwn1_ref, pe_ref, acc_ref):
  i = pl.program_id(0)
  x = wni_ref[...]
  hpe = _bnrelu3(_dot3(pe0_ref, x[9:12]), bnpe0_ref)
  hwn = _bnrelu3(_dot3(wn0_ref, x), bnwn0_ref)
  hpe1 = _bnrelu3(_dot3(pe1_ref, hpe), bnpe1_ref)
  pe_ref[...] = jnp.max(hpe1, axis=1)
  hwn1 = _bnrelu3(_dot3(wn1_ref, hwn), bnwn1_ref)
  wn2 = _dot3(wn2_ref, hwn1)
  mask = _edge_mask(i)
  s, q = _stat3(wn2, mask)
  _acc(acc_ref, jnp.concatenate([s, q], axis=0), i)


def _p4_body(wni_ref, gf_ref, linw_ref, wn0_ref, wn1_ref, wn2_ref, bnwn0_ref,
             bnwn1_ref, bnwn2_ref, linpre_ref, acc_ref, pc_ref):
  i = pl.program_id(0)
  x = wni_ref[...]
  hwn = _bnrelu3(_dot3(wn0_ref, x), bnwn0_ref)
  hwn1 = _bnrelu3(_dot3(wn1_ref, hwn), bnwn1_ref)
  wpl = _bnrelu3(_dot3(wn2_ref, hwn1), bnwn2_ref)

  gfs = [gf_ref[k].astype(f32) for k in range(K)]
  for d in range(16):
    a = gfs[0] * jnp.broadcast_to(wpl[d, 0][None, :], (64, T))
    for k in range(1, K):
      a = a + gfs[k] * jnp.broadcast_to(wpl[d, k][None, :], (64, T))
    pc_ref[d] = a

  pc = pc_ref[...].reshape(1024, T)
  linpre = lax.dot_general(linw_ref[...], pc, (((1,), (0,)), ((), ())),
                           preferred_element_type=f32)
  linpre_ref[...] = linpre
  mask = (i * T + lax.broadcasted_iota(jnp.int32, (64, T), 1)) < N
  s = _lanefold(jnp.where(mask, linpre, 0.0))
  sq = _lanefold(jnp.where(mask, linpre * linpre, 0.0))
  _acc(acc_ref, jnp.concatenate([s, sq], axis=0), i)


def _p5_body(linpre_ref, pe_ref, lm_ref, ls_ref, m2w_ref, acc_ref):
  i = pl.program_id(0)
  h = jnp.maximum((linpre_ref[...] - lm_ref[...]) * ls_ref[...], 0.0)
  cat = jnp.concatenate([h, pe_ref[...]], axis=0)
  m2 = lax.dot_general(m2w_ref[...], cat, (((1,), (0,)), ((), ())),
                       preferred_element_type=f32)
  mask = (i * T + lax.broadcasted_iota(jnp.int32, (64, T), 1)) < N
  s = _lanefold(jnp.where(mask, m2, 0.0))
  sq = _lanefold(jnp.where(mask, m2 * m2, 0.0))
  _acc(acc_ref, jnp.concatenate([s, sq], axis=0), i)


def _p6_body(linpre_ref, pe_ref, lm_ref, ls_ref, mm_ref, ms_ref, m2w_ref,
             df_ref, out_ref):
  h = jnp.maximum((linpre_ref[...] - lm_ref[...]) * ls_ref[...], 0.0)
  cat = jnp.concatenate([h, pe_ref[...]], axis=0)
  m2 = lax.dot_general(m2w_ref[...], cat, (((1,), (0,)), ((), ())),
                       preferred_element_type=f32)
  y = (m2 - mm_ref[...]) * ms_ref[...]
  y = jnp.where(y >= 0.0, y, 0.1 * y)
  out_ref[...] = y + df_ref[...]


def _vspec(shape, imap):
  return pl.BlockSpec(shape, imap)


def _full2(shape):
  return pl.BlockSpec(shape, lambda i: (0, 0))


def _mvinv(sums, sumsqs, cnt):
  m = sums / cnt
  v = sumsqs / cnt - m * m
  return m, 1.0 / jnp.sqrt(v + 1e-5)


def kernel(sparse_xyz, sparse_feats, dense_xyz, dense_feats, dense_xyz_norm,
           sparse_xyz_norm, wn0_W, wn0_b, wn1_W, wn1_b, wn2_W, wn2_b, pe0_W,
           pe0_b, pe1_W, pe1_b, lin_W, lin_b, mlp2_W, mlp2_b, nei_inds):
  sx = sparse_xyz[0]
  sn = sparse_xyz_norm[0]
  sf = sparse_feats[0]
  tableg = jnp.concatenate([sx, sn, jnp.zeros((M, 10), f32)], axis=1)
  nip = jnp.pad(nei_inds[0].astype(jnp.int32).T, ((0, 0), (0, NP - N)))
  idx3 = nip.reshape(NW, CH, CHUNK)

  gg = _sc_gather(tableg, idx3, 16)
  gf = _sc_gather(sf.astype(jnp.bfloat16), idx3, 64, jnp.bfloat16)

  gt = gg.T.reshape(16, K, NP)
  gft = gf.reshape(K, NP, 64).transpose(0, 2, 1)
  dxt = jnp.pad(dense_xyz[0].T, ((0, 0), (0, NP - N)))
  dnt = jnp.pad(dense_xyz_norm[0].T, ((0, 0), (0, NP - N)))
  dft = jnp.pad(dense_feats[0].T, ((0, 0), (0, NP - N)))
  lin2 = lin_W.reshape(64, 64, 16).transpose(0, 2, 1).reshape(64, 1024)

  wni, acc1 = pl.pallas_call(
      _p1_body,
      grid=(GRID,),
      in_specs=[
          _vspec((6, K, T), lambda i: (0, 0, i)),
          _vspec((3, T), lambda i: (0, i)),
          _vspec((3, T), lambda i: (0, i)),
          _full2((16, 3)),
          _full2((8, 12)),
      ],
      out_specs=[
          _vspec((12, K, T), lambda i: (0, 0, i)),
          _full2((48, 128)),
      ],
      out_shape=[
          jax.ShapeDtypeStruct((12, K, NP), f32),
          jax.ShapeDtypeStruct((48, 128), f32),
      ],
  )(gt, dxt, dnt, pe0_W, wn0_W)
  s1 = jnp.sum(acc1, axis=1)
  pe0_m, pe0_s = _mvinv(s1[0:16], s1[24:40], CNT_E)
  wn0_m, wn0_s = _mvinv(s1[16:24], s1[40:48], CNT_E)
  bnpe0 = jnp.stack([pe0_m, pe0_s])
  bnwn0 = jnp.stack([wn0_m, wn0_s])

  acc2 = pl.pallas_call(
      _p2_body,
      grid=(GRID,),
      in_specs=[
          _vspec((12, K, T), lambda i: (0, 0, i)),
          _full2((16, 3)),
          _full2((8, 12)),
          _full2((16, 16)),
          _full2((8, 8)),
          _full2((2, 16)),
          _full2((2, 8)),
      ],
      out_specs=[_full2((48, 128))],
      out_shape=[jax.ShapeDtypeStruct((48, 128), f32)],
  )(wni, pe0_W, wn0_W, pe1_W, wn1_W, bnpe0, bnwn0)[0]
  s2 = jnp.sum(acc2, axis=1)
  pe1_m, pe1_s = _mvinv(s2[0:16], s2[24:40], CNT_E)
  wn1_m, wn1_s = _mvinv(s2[16:24], s2[40:48], CNT_E)
  bnpe1 = jnp.stack([pe1_m, pe1_s])
  bnwn1 = jnp.stack([wn1_m, wn1_s])

  pet, acc3 = pl.pallas_call(
      _p3_body,
      grid=(GRID,),
      in_specs=[
          _vspec((12, K, T), lambda i: (0, 0, i)),
          _full2((16, 3)),
          _full2((8, 12)),
          _full2((16, 16)),
          _full2((8, 8)),
          _full2((16, 8)),
          _full2((2, 16)),
          _full2((2, 8)),
          _full2((2, 16)),
          _full2((2, 8)),
      ],
      out_specs=[
          _vspec((16, T), lambda i: (0, i)),
          _full2((32, 128)),
      ],
      out_shape=[
          jax.ShapeDtypeStruct((16, NP), f32),
          jax.ShapeDtypeStruct((32, 128), f32),
      ],
  )(wni, pe0_W, wn0_W, pe1_W, wn1_W, wn2_W, bnpe0, bnwn0, bnpe1, bnwn1)
  s3 = jnp.sum(acc3, axis=1)
  wn2_m, wn2_s = _mvinv(s3[0:16], s3[16:32], CNT_E)
  bnwn2 = jnp.stack([wn2_m, wn2_s])

  linpret, acc4 = pl.pallas_call(
      _p4_body,
      grid=(GRID,),
      in_specs=[
          _vspec((12, K, T), lambda i: (0, 0, i)),
          _vspec((K, 64, T), lambda i: (0, 0, i)),
          _full2((64, 1024)),
          _full2((8, 12)),
          _full2((8, 8)),
          _full2((16, 8)),
          _full2((2, 8)),
          _full2((2, 8)),
          _full2((2, 16)),
      ],
      out_specs=[
          _vspec((64, T), lambda i: (0, i)),
          _full2((128, 128)),
      ],
      out_shape=[
          jax.ShapeDtypeStruct((64, NP), f32),
          jax.ShapeDtypeStruct((128, 128), f32),
      ],
      scratch_shapes=[pltpu.VMEM((16, 64, T), f32)],
  )(wni, gft, lin2, wn0_W, wn1_W, wn2_W, bnwn0, bnwn1, bnwn2)
  s4 = jnp.sum(acc4, axis=1)
  lin_m, lin_s = _mvinv(s4[0:64], s4[64:128], CNT_N)
  lmc = lin_m.reshape(64, 1)
  lsc = lin_s.reshape(64, 1)

  acc5 = pl.pallas_call(
      _p5_body,
      grid=(GRID,),
      in_specs=[
          _vspec((64, T), lambda i: (0, i)),
          _vspec((16, T), lambda i: (0, i)),
          _full2((64, 1)),
          _full2((64, 1)),
          _full2((64, 80)),
      ],
      out_specs=[_full2((128, 128))],
      out_shape=[jax.ShapeDtypeStruct((128, 128), f32)],
  )(linpret, pet, lmc, lsc, mlp2_W)[0]
  s5 = jnp.sum(acc5, axis=1)
  m2_m, m2_s = _mvinv(s5[0:64], s5[64:128], CNT_N)
  mmc = m2_m.reshape(64, 1)
  msc = m2_s.reshape(64, 1)

  outp = pl.pallas_call(
      _p6_body,
      grid=(GRID,),
      in_specs=[
          _vspec((64, T), lambda i: (0, i)),
          _vspec((16, T), lambda i: (0, i)),
          _full2((64, 1)),
          _full2((64, 1)),
          _full2((64, 1)),
          _full2((64, 1)),
          _full2((64, 80)),
          _vspec((64, T), lambda i: (0, i)),
      ],
      out_specs=[_vspec((64, T), lambda i: (0, i))],
      out_shape=[jax.ShapeDtypeStruct((64, NP), f32)],
  )(linpret, pet, lmc, lsc, mmc, msc, mlp2_W, dft)[0]

  return outp[:, :N].T.reshape(1, N, 64)


# speedup vs baseline: 1.0485x; 1.0485x over previous
import functools

import jax
import jax.numpy as jnp
from jax import lax
from jax.experimental import pallas as pl
from jax.experimental.pallas import tpu as pltpu
from jax.experimental.pallas import tpu_sc as plsc

f32 = jnp.float32

N = 50000
M = 12500
K = 16
NP = 50176
E = K * NP
NW = 32
CHUNK = 128
CH = E // (NW * CHUNK)
T = 1024
GRID = NP // T
CNT_E = float(N * K)
CNT_N = float(N)


def _sc_gather(table, idx3, D):
  mesh = plsc.VectorSubcoreMesh(core_axis_name="c", subcore_axis_name="s")

  @functools.partial(
      pl.kernel,
      out_type=jax.ShapeDtypeStruct((E, D), f32),
      mesh=mesh,
      compiler_params=pltpu.CompilerParams(use_tc_tiling_on_sc=False),
      scratch_types=[
          pltpu.VMEM((CH, CHUNK), jnp.int32),
          pltpu.VMEM((CHUNK, D), f32),
          pltpu.VMEM((CHUNK, D), f32),
          pltpu.SemaphoreType.DMA,
          pltpu.SemaphoreType.DMA,
          pltpu.SemaphoreType.DMA,
          pltpu.SemaphoreType.DMA,
      ],
  )
  def gk(table_hbm, idx_hbm, out_hbm, idx_v, rows0, rows1, g0, g1, w0, w1):
    wid = lax.axis_index("s") * 2 + lax.axis_index("c")
    pltpu.sync_copy(idx_hbm.at[wid], idx_v)

    def body(i, carry):
      j0 = i * 2
      j1 = j0 + 1
      c0 = pltpu.async_copy(table_hbm.at[idx_v.at[j0]], rows0, g0)
      c1 = pltpu.async_copy(table_hbm.at[idx_v.at[j1]], rows1, g1)
      c0.wait()
      wc0 = pltpu.async_copy(
          rows0, out_hbm.at[pl.ds((wid * CH + j0) * CHUNK, CHUNK)], w0)
      c1.wait()
      wc1 = pltpu.async_copy(
          rows1, out_hbm.at[pl.ds((wid * CH + j1) * CHUNK, CHUNK)], w1)
      wc0.wait()
      wc1.wait()
      return carry

    lax.fori_loop(0, CH // 2, body, 0)

  return gk(table, idx3)


def _dot3(w_ref, x):
  return lax.dot_general(w_ref[...], x, (((1,), (0,)), ((), ())),
                         preferred_element_type=f32)


def _bnrelu3(x, bn_ref):
  c = x.shape[0]
  m = bn_ref[0].reshape(c, 1, 1)
  s = bn_ref[1].reshape(c, 1, 1)
  return jnp.maximum((x - m) * s, 0.0)


def _lanefold(x):
  r = x[..., 0:128]
  for o in range(128, x.shape[-1], 128):
    r = r + x[..., o:o + 128]
  return r


def _stat3(x, mask):
  xm = jnp.where(mask[None, :, :], x, 0.0)
  s = jnp.sum(_lanefold(xm), axis=1)
  xq = jnp.where(mask[None, :, :], x * x, 0.0)
  q = jnp.sum(_lanefold(xq), axis=1)
  return s, q


def _acc(acc_ref, part, i):
  @pl.when(i == 0)
  def _():
    acc_ref[...] = jnp.zeros_like(acc_ref[...])
  acc_ref[...] = acc_ref[...] + part


def _edge_mask(i):
  col = i * T + lax.broadcasted_iota(jnp.int32, (K, T), 1)
  return col < N


def _vi(loc, gn, q):
  rn = jnp.sqrt(loc[0] * loc[0] + loc[1] * loc[1] + loc[2] * loc[2])
  rinv = 1.0 / jnp.maximum(rn, 1e-12)
  rh = [l * rinv for l in loc]
  th2 = q[0] * rh[0] + q[1] * rh[1] + q[2] * rh[2]
  v = [q[c] - th2 * rh[c] for c in range(3)]
  vn = jnp.sqrt(v[0] * v[0] + v[1] * v[1] + v[2] * v[2])
  vinv = 1.0 / jnp.maximum(vn, 1e-12)
  vm = [x * vinv for x in v]
  w = [rh[1] * vm[2] - rh[2] * vm[1],
       rh[2] * vm[0] - rh[0] * vm[2],
       rh[0] * vm[1] - rh[1] * vm[0]]
  wnn = jnp.sqrt(w[0] * w[0] + w[1] * w[1] + w[2] * w[2])
  winv = 1.0 / jnp.maximum(wnn, 1e-12)
  wm = [x * winv for x in w]
  th1 = gn[0] * q[0] + gn[1] * q[1] + gn[2] * q[2]
  th3 = rh[0] * gn[0] + rh[1] * gn[1] + rh[2] * gn[2]
  th4 = loc[0] * q[0] + loc[1] * q[1] + loc[2] * q[2]
  th6 = gn[0] * vm[0] + gn[1] * vm[1] + gn[2] * vm[2]
  th7 = gn[0] * wm[0] + gn[1] * wm[1] + gn[2] * wm[2]
  cgq = [gn[1] * q[2] - gn[2] * q[1],
         gn[2] * q[0] - gn[0] * q[2],
         gn[0] * q[1] - gn[1] * q[0]]
  th8 = loc[0] * cgq[0] + loc[1] * cgq[1] + loc[2] * cgq[2]
  return [th1, th2, th3, th4, th3, th6, th7, th8, rn, loc[0], loc[1], loc[2]]


def _p1_body(g_ref, dx_ref, dn_ref, pe0_ref, wn0_ref, wni_ref, acc_ref):
  i = pl.program_id(0)
  dx = [jnp.broadcast_to(dx_ref[c][None, :], (K, T)) for c in range(3)]
  dn = [jnp.broadcast_to(dn_ref[c][None, :], (K, T)) for c in range(3)]
  loc = [g_ref[c] - dx[c] for c in range(3)]
  gn = [g_ref[3 + c] for c in range(3)]
  wni = _vi(loc, gn, dn)
  for c in range(12):
    wni_ref[c] = wni[c]
  x = wni_ref[...]
  pe0 = _dot3(pe0_ref, x[9:12])
  wn0 = _dot3(wn0_ref, x)
  mask = _edge_mask(i)
  s_pe, q_pe = _stat3(pe0, mask)
  s_wn, q_wn = _stat3(wn0, mask)
  _acc(acc_ref, jnp.concatenate([s_pe, s_wn, q_pe, q_wn], axis=0), i)


def _p2_body(wni_ref, pe0_ref, wn0_ref, pe1_ref, wn1_ref, bnpe0_ref,
             bnwn0_ref, acc_ref):
  i = pl.program_id(0)
  x = wni_ref[...]
  hpe = _bnrelu3(_dot3(pe0_ref, x[9:12]), bnpe0_ref)
  hwn = _bnrelu3(_dot3(wn0_ref, x), bnwn0_ref)
  pe1 = _dot3(pe1_ref, hpe)
  wn1 = _dot3(wn1_ref, hwn)
  mask = _edge_mask(i)
  s_pe, q_pe = _stat3(pe1, mask)
  s_wn, q_wn = _stat3(wn1, mask)
  _acc(acc_ref, jnp.concatenate([s_pe, s_wn, q_pe, q_wn], axis=0), i)


def _p3_body(wni_ref, pe0_ref, wn0_ref, pe1_ref, wn1_ref, wn2_ref, bnpe0_ref,
             bnwn0_ref, bnpe1_ref, bnwn1_ref, pe_ref, acc_ref):
  i = pl.program_id(0)
  x = wni_ref[...]
  hpe = _bnrelu3(_dot3(pe0_ref, x[9:12]), bnpe0_ref)
  hwn = _bnrelu3(_dot3(wn0_ref, x), bnwn0_ref)
  hpe1 = _bnrelu3(_dot3(pe1_ref, hpe), bnpe1_ref)
  pe_ref[...] = jnp.max(hpe1, axis=1)
  hwn1 = _bnrelu3(_dot3(wn1_ref, hwn), bnwn1_ref)
  wn2 = _dot3(wn2_ref, hwn1)
  mask = _edge_mask(i)
  s, q = _stat3(wn2, mask)
  _acc(acc_ref, jnp.concatenate([s, q], axis=0), i)


def _p4_body(wni_ref, gf_ref, linw_ref, wn0_ref, wn1_ref, wn2_ref, bnwn0_ref,
             bnwn1_ref, bnwn2_ref, linpre_ref, acc_ref, pc_ref):
  i = pl.program_id(0)
  x = wni_ref[...]
  hwn = _bnrelu3(_dot3(wn0_ref, x), bnwn0_ref)
  hwn1 = _bnrelu3(_dot3(wn1_ref, hwn), bnwn1_ref)
  wpl = _bnrelu3(_dot3(wn2_ref, hwn1), bnwn2_ref)

  for d in range(16):
    a = gf_ref[0] * jnp.broadcast_to(wpl[d, 0][None, :], (64, T))
    for k in range(1, K):
      a = a + gf_ref[k] * jnp.broadcast_to(wpl[d, k][None, :], (64, T))
    pc_ref[d] = a

  pc = pc_ref[...].reshape(1024, T)
  linpre = lax.dot_general(linw_ref[...], pc, (((1,), (0,)), ((), ())),
                           preferred_element_type=f32)
  linpre_ref[...] = linpre
  mask = (i * T + lax.broadcasted_iota(jnp.int32, (64, T), 1)) < N
  s = _lanefold(jnp.where(mask, linpre, 0.0))
  sq = _lanefold(jnp.where(mask, linpre * linpre, 0.0))
  _acc(acc_ref, jnp.concatenate([s, sq], axis=0), i)


def _p5_body(linpre_ref, pe_ref, lm_ref, ls_ref, m2w_ref, acc_ref):
  i = pl.program_id(0)
  h = jnp.maximum((linpre_ref[...] - lm_ref[...]) * ls_ref[...], 0.0)
  cat = jnp.concatenate([h, pe_ref[...]], axis=0)
  m2 = lax.dot_general(m2w_ref[...], cat, (((1,), (0,)), ((), ())),
                       preferred_element_type=f32)
  mask = (i * T + lax.broadcasted_iota(jnp.int32, (64, T), 1)) < N
  s = _lanefold(jnp.where(mask, m2, 0.0))
  sq = _lanefold(jnp.where(mask, m2 * m2, 0.0))
  _acc(acc_ref, jnp.concatenate([s, sq], axis=0), i)


def _p6_body(linpre_ref, pe_ref, lm_ref, ls_ref, mm_ref, ms_ref, m2w_ref,
             df_ref, out_ref):
  h = jnp.maximum((linpre_ref[...] - lm_ref[...]) * ls_ref[...], 0.0)
  cat = jnp.concatenate([h, pe_ref[...]], axis=0)
  m2 = lax.dot_general(m2w_ref[...], cat, (((1,), (0,)), ((), ())),
                       preferred_element_type=f32)
  y = (m2 - mm_ref[...]) * ms_ref[...]
  y = jnp.where(y >= 0.0, y, 0.1 * y)
  out_ref[...] = y + df_ref[...]


def _vspec(shape, imap):
  return pl.BlockSpec(shape, imap)


def _full2(shape):
  return pl.BlockSpec(shape, lambda i: (0, 0))


def _mvinv(sums, sumsqs, cnt):
  m = sums / cnt
  v = sumsqs / cnt - m * m
  return m, 1.0 / jnp.sqrt(v + 1e-5)


def kernel(sparse_xyz, sparse_feats, dense_xyz, dense_feats, dense_xyz_norm,
           sparse_xyz_norm, wn0_W, wn0_b, wn1_W, wn1_b, wn2_W, wn2_b, pe0_W,
           pe0_b, pe1_W, pe1_b, lin_W, lin_b, mlp2_W, mlp2_b, nei_inds):
  sx = sparse_xyz[0]
  sn = sparse_xyz_norm[0]
  sf = sparse_feats[0]
  tableg = jnp.concatenate([sx, sn, jnp.zeros((M, 10), f32)], axis=1)
  nip = jnp.pad(nei_inds[0].astype(jnp.int32).T, ((0, 0), (0, NP - N)))
  idx3 = nip.reshape(NW, CH, CHUNK)

  gg = _sc_gather(tableg, idx3, 16)
  gf = _sc_gather(sf, idx3, 64)

  gt = gg.T.reshape(16, K, NP)
  gft = gf.reshape(K, NP, 64).transpose(0, 2, 1)
  dxt = jnp.pad(dense_xyz[0].T, ((0, 0), (0, NP - N)))
  dnt = jnp.pad(dense_xyz_norm[0].T, ((0, 0), (0, NP - N)))
  dft = jnp.pad(dense_feats[0].T, ((0, 0), (0, NP - N)))
  lin2 = lin_W.reshape(64, 64, 16).transpose(0, 2, 1).reshape(64, 1024)

  wni, acc1 = pl.pallas_call(
      _p1_body,
      grid=(GRID,),
      in_specs=[
          _vspec((6, K, T), lambda i: (0, 0, i)),
          _vspec((3, T), lambda i: (0, i)),
          _vspec((3, T), lambda i: (0, i)),
          _full2((16, 3)),
          _full2((8, 12)),
      ],
      out_specs=[
          _vspec((12, K, T), lambda i: (0, 0, i)),
          _full2((48, 128)),
      ],
      out_shape=[
          jax.ShapeDtypeStruct((12, K, NP), f32),
          jax.ShapeDtypeStruct((48, 128), f32),
      ],
  )(gt, dxt, dnt, pe0_W, wn0_W)
  s1 = jnp.sum(acc1, axis=1)
  pe0_m, pe0_s = _mvinv(s1[0:16], s1[24:40], CNT_E)
  wn0_m, wn0_s = _mvinv(s1[16:24], s1[40:48], CNT_E)
  bnpe0 = jnp.stack([pe0_m, pe0_s])
  bnwn0 = jnp.stack([wn0_m, wn0_s])

  acc2 = pl.pallas_call(
      _p2_body,
      grid=(GRID,),
      in_specs=[
          _vspec((12, K, T), lambda i: (0, 0, i)),
          _full2((16, 3)),
          _full2((8, 12)),
          _full2((16, 16)),
          _full2((8, 8)),
          _full2((2, 16)),
          _full2((2, 8)),
      ],
      out_specs=[_full2((48, 128))],
      out_shape=[jax.ShapeDtypeStruct((48, 128), f32)],
  )(wni, pe0_W, wn0_W, pe1_W, wn1_W, bnpe0, bnwn0)[0]
  s2 = jnp.sum(acc2, axis=1)
  pe1_m, pe1_s = _mvinv(s2[0:16], s2[24:40], CNT_E)
  wn1_m, wn1_s = _mvinv(s2[16:24], s2[40:48], CNT_E)
  bnpe1 = jnp.stack([pe1_m, pe1_s])
  bnwn1 = jnp.stack([wn1_m, wn1_s])

  pet, acc3 = pl.pallas_call(
      _p3_body,
      grid=(GRID,),
      in_specs=[
          _vspec((12, K, T), lambda i: (0, 0, i)),
          _full2((16, 3)),
          _full2((8, 12)),
          _full2((16, 16)),
          _full2((8, 8)),
          _full2((16, 8)),
          _full2((2, 16)),
          _full2((2, 8)),
          _full2((2, 16)),
          _full2((2, 8)),
      ],
      out_specs=[
          _vspec((16, T), lambda i: (0, i)),
          _full2((32, 128)),
      ],
      out_shape=[
          jax.ShapeDtypeStruct((16, NP), f32),
          jax.ShapeDtypeStruct((32, 128), f32),
      ],
  )(wni, pe0_W, wn0_W, pe1_W, wn1_W, wn2_W, bnpe0, bnwn0, bnpe1, bnwn1)
  s3 = jnp.sum(acc3, axis=1)
  wn2_m, wn2_s = _mvinv(s3[0:16], s3[16:32], CNT_E)
  bnwn2 = jnp.stack([wn2_m, wn2_s])

  linpret, acc4 = pl.pallas_call(
      _p4_body,
      grid=(GRID,),
      in_specs=[
          _vspec((12, K, T), lambda i: (0, 0, i)),
          _vspec((K, 64, T), lambda i: (0, 0, i)),
          _full2((64, 1024)),
          _full2((8, 12)),
          _full2((8, 8)),
          _full2((16, 8)),
          _full2((2, 8)),
          _full2((2, 8)),
          _full2((2, 16)),
      ],
      out_specs=[
          _vspec((64, T), lambda i: (0, i)),
          _full2((128, 128)),
      ],
      out_shape=[
          jax.ShapeDtypeStruct((64, NP), f32),
          jax.ShapeDtypeStruct((128, 128), f32),
      ],
      scratch_shapes=[pltpu.VMEM((16, 64, T), f32)],
  )(wni, gft, lin2, wn0_W, wn1_W, wn2_W, bnwn0, bnwn1, bnwn2)
  s4 = jnp.sum(acc4, axis=1)
  lin_m, lin_s = _mvinv(s4[0:64], s4[64:128], CNT_N)
  lmc = lin_m.reshape(64, 1)
  lsc = lin_s.reshape(64, 1)

  acc5 = pl.pallas_call(
      _p5_body,
      grid=(GRID,),
      in_specs=[
          _vspec((64, T), lambda i: (0, i)),
          _vspec((16, T), lambda i: (0, i)),
          _full2((64, 1)),
          _full2((64, 1)),
          _full2((64, 80)),
      ],
      out_specs=[_full2((128, 128))],
      out_shape=[jax.ShapeDtypeStruct((128, 128), f32)],
  )(linpret, pet, lmc, lsc, mlp2_W)[0]
  s5 = jnp.sum(acc5, axis=1)
  m2_m, m2_s = _mvinv(s5[0:64], s5[64:128], CNT_N)
  mmc = m2_m.reshape(64, 1)
  msc = m2_s.reshape(64, 1)

  outp = pl.pallas_call(
      _p6_body,
      grid=(GRID,),
      in_specs=[
          _vspec((64, T), lambda i: (0, i)),
          _vspec((16, T), lambda i: (0, i)),
          _full2((64, 1)),
          _full2((64, 1)),
          _full2((64, 1)),
          _full2((64, 1)),
          _full2((64, 80)),
          _vspec((64, T), lambda i: (0, i)),
      ],
      out_specs=[_vspec((64, T), lambda i: (0, i))],
      out_shape=[jax.ShapeDtypeStruct((64, NP), f32)],
  )(linpret, pet, lmc, lsc, mmc, msc, mlp2_W, dft)[0]

  return outp[:, :N].T.reshape(1, N, 64)
